# baseline (speedup 1.0000x reference)
import functools
import math

import jax
import jax.numpy as jnp
from jax.experimental import pallas as pl
from jax.experimental.pallas import tpu as pltpu

K_SLOTS = 8192
D = 256
LOG_K = math.log(K_SLOTS)
NEG_BIG = -1e30

BN = 512
BK = 1024


def _vq_body(z2_ref, cb_ref, csq_ref, zsq_ref, g_ref,
             zq_ref, idx_ref, kl_ref, cl_ref,
             m1, s1, t1, m2, s2, acc, bv, bi):
    i = pl.program_id(0)
    j = pl.program_id(1)
    nk = pl.num_programs(1)

    @pl.when(j == 0)
    def _init():
        m1[...] = jnp.full_like(m1, NEG_BIG)
        s1[...] = jnp.zeros_like(s1)
        t1[...] = jnp.zeros_like(t1)
        m2[...] = jnp.full_like(m2, NEG_BIG)
        s2[...] = jnp.zeros_like(s2)
        acc[...] = jnp.zeros_like(acc)
        bv[...] = jnp.full_like(bv, NEG_BIG)
        bi[...] = jnp.zeros_like(bi)

    z2 = z2_ref[...]
    cb = cb_ref[pl.ds(j * BK, BK), :]
    g = g_ref[...]

    z_sqr = zsq_ref[...]
    c_sqr = csq_ref[...]
    zc2 = jax.lax.dot_general(
        z2, cb, (((1,), (1,)), ((), ())),
        preferred_element_type=jnp.float32)

    l = zc2 - (z_sqr + c_sqr)
    L = l + g

    bm1 = jnp.max(l, axis=1, keepdims=True)
    m1o = m1[...]
    m1n = jnp.maximum(m1o, bm1)
    a1 = jnp.exp(m1o - m1n)
    e1 = jnp.exp(l - m1n)
    s1o = s1[...]
    s1[...] = a1 * s1o + jnp.sum(e1, axis=1, keepdims=True)
    t1[...] = a1 * (t1[...] + s1o * (m1o - m1n)) + jnp.sum(
        e1 * (l - m1n), axis=1, keepdims=True)
    m1[...] = m1n

    bm2 = jnp.max(L, axis=1, keepdims=True)
    m2o = m2[...]
    m2n = jnp.maximum(m2o, bm2)
    a2 = jnp.exp((m2o - m2n) * 2.0)
    e2 = jnp.exp((L - m2n) * 2.0)
    s2[...] = a2 * s2[...] + jnp.sum(e2, axis=1, keepdims=True)
    acc[...] = a2 * acc[...] + jax.lax.dot_general(
        e2, cb, (((1,), (0,)), ((), ())),
        preferred_element_type=jnp.float32)
    m2[...] = m2n

    col = jax.lax.broadcasted_iota(jnp.int32, (BN, BK), 1)
    idx_blk = jnp.min(jnp.where(L == bm2, col, K_SLOTS),
                      axis=1, keepdims=True) + j * BK
    bvo = bv[...]
    take = bm2 > bvo
    bi[...] = jnp.where(take, idx_blk, bi[...])
    bv[...] = jnp.maximum(bm2, bvo)

    @pl.when(j == nk - 1)
    def _fin():
        s1f = s1[...]
        r = t1[...] / s1f
        kl_rows = r - jnp.log(s1f) + LOG_K
        cl_rows = -(r + m1[...])
        zq_ref[...] = acc[...] / s2[...]
        idx_ref[...] = bi[...]

        @pl.when(i == 0)
        def _zero():
            kl_ref[...] = jnp.zeros_like(kl_ref)
            cl_ref[...] = jnp.zeros_like(cl_ref)

        kl_ref[...] += jnp.sum(kl_rows, keepdims=True) / 8.0
        cl_ref[...] += jnp.sum(cl_rows, keepdims=True) / 8.0


@functools.partial(jax.jit, static_argnames=())
def _vq_call(z, codebook, gumbel):
    n = z.shape[0]
    c_sqr = jnp.sum(codebook ** 2, axis=1)[None, :]
    z_sqr = jnp.sum(z ** 2, axis=1, keepdims=True)
    z2 = z + z
    grid = (n // BN, K_SLOTS // BK)
    zq, idx, kl, cl = pl.pallas_call(
        _vq_body,
        grid=grid,
        in_specs=[
            pl.BlockSpec((BN, D), lambda i, j: (i, 0)),
            pl.BlockSpec((K_SLOTS, D), lambda i, j: (0, 0)),
            pl.BlockSpec((1, BK), lambda i, j: (0, j)),
            pl.BlockSpec((BN, 1), lambda i, j: (i, 0)),
            pl.BlockSpec((BN, BK), lambda i, j: (i, j)),
        ],
        out_specs=[
            pl.BlockSpec((BN, D), lambda i, j: (i, 0)),
            pl.BlockSpec((BN, 1), lambda i, j: (i, 0)),
            pl.BlockSpec((1, 1), lambda i, j: (0, 0)),
            pl.BlockSpec((1, 1), lambda i, j: (0, 0)),
        ],
        out_shape=[
            jax.ShapeDtypeStruct((n, D), jnp.float32),
            jax.ShapeDtypeStruct((n, 1), jnp.int32),
            jax.ShapeDtypeStruct((1, 1), jnp.float32),
            jax.ShapeDtypeStruct((1, 1), jnp.float32),
        ],
        scratch_shapes=[
            pltpu.VMEM((BN, 1), jnp.float32),
            pltpu.VMEM((BN, 1), jnp.float32),
            pltpu.VMEM((BN, 1), jnp.float32),
            pltpu.VMEM((BN, 1), jnp.float32),
            pltpu.VMEM((BN, 1), jnp.float32),
            pltpu.VMEM((BN, D), jnp.float32),
            pltpu.VMEM((BN, 1), jnp.float32),
            pltpu.VMEM((BN, 1), jnp.int32),
        ],
    )(z2, codebook, c_sqr, z_sqr, gumbel)
    return zq, idx, kl, cl


def kernel(z_e, codebook, gumbel):
    bs, feat, w, h = z_e.shape
    z = jnp.transpose(z_e, (0, 2, 3, 1)).reshape(bs * w * h, feat)
    zq_flat, idx_flat, kl, cl = _vq_call(z, codebook, gumbel)
    z_q = zq_flat.reshape(bs, w, h, feat).transpose(0, 3, 1, 2)
    hard = idx_flat.reshape(bs, w, h)
    return z_q, hard, kl[0, 0], cl[0, 0]


# speedup vs baseline: 1.0400x; 1.0400x over previous
import functools
import math

import jax
import jax.numpy as jnp
from jax.experimental import pallas as pl
from jax.experimental.pallas import tpu as pltpu

K_SLOTS = 8192
D = 256
LOG_K = math.log(K_SLOTS)
NEG_BIG = -1e30

BN = 512
BK = 1024


def _vq_body(z2_ref, cb_ref, csq_ref, zsq_ref, g_ref,
             zq_ref, idx_ref, kl_ref, cl_ref,
             s1, t1, m2, s2, acc, bv, bi):
    i = pl.program_id(0)
    j = pl.program_id(1)
    nk = pl.num_programs(1)

    @pl.when(j == 0)
    def _init():
        s1[...] = jnp.zeros_like(s1)
        t1[...] = jnp.zeros_like(t1)
        m2[...] = jnp.full_like(m2, NEG_BIG)
        s2[...] = jnp.zeros_like(s2)
        acc[...] = jnp.zeros_like(acc)
        bv[...] = jnp.full_like(bv, NEG_BIG)
        bi[...] = jnp.zeros_like(bi)

    z2 = z2_ref[...]
    cb = cb_ref[...]
    g = g_ref[...]

    z_sqr = zsq_ref[...]
    c_sqr = csq_ref[...]
    zc2 = jax.lax.dot_general(
        z2, cb, (((1,), (1,)), ((), ())),
        preferred_element_type=jnp.float32)

    l = zc2 - (z_sqr + c_sqr)
    L = l + g

    bm2 = jnp.max(L, axis=1, keepdims=True)
    m2o = m2[...]
    m2n = jnp.maximum(m2o, bm2)
    dm = m2o - m2n
    a1 = jnp.exp(dm)
    a2 = a1 * a1

    u1 = l - m2n
    e1 = jnp.exp(u1)
    s1o = s1[...]
    s1[...] = a1 * s1o + jnp.sum(e1, axis=1, keepdims=True)
    t1[...] = a1 * (t1[...] + s1o * dm) + jnp.sum(
        e1 * u1, axis=1, keepdims=True)

    e2 = jnp.exp((L - m2n) * 2.0)
    s2[...] = a2 * s2[...] + jnp.sum(e2, axis=1, keepdims=True)
    acc[...] = a2 * acc[...] + jax.lax.dot_general(
        e2, cb, (((1,), (0,)), ((), ())),
        preferred_element_type=jnp.float32)
    m2[...] = m2n

    col = jax.lax.broadcasted_iota(jnp.int32, (BN, BK), 1)
    idx_blk = jnp.min(jnp.where(L == bm2, col, K_SLOTS),
                      axis=1, keepdims=True) + j * BK
    bvo = bv[...]
    take = bm2 > bvo
    bi[...] = jnp.where(take, idx_blk, bi[...])
    bv[...] = jnp.maximum(bm2, bvo)

    @pl.when(j == nk - 1)
    def _fin():
        s1f = s1[...]
        r = t1[...] / s1f
        kl_rows = r - jnp.log(s1f) + LOG_K
        cl_rows = -(r + m2[...])
        zq_ref[...] = acc[...] / s2[...]
        idx_ref[...] = bi[...]

        @pl.when(i == 0)
        def _zero():
            kl_ref[...] = jnp.zeros_like(kl_ref)
            cl_ref[...] = jnp.zeros_like(cl_ref)

        kl_ref[...] += jnp.sum(kl_rows, keepdims=True) / 8.0
        cl_ref[...] += jnp.sum(cl_rows, keepdims=True) / 8.0


@functools.partial(jax.jit, static_argnames=())
def _vq_call(z, codebook, gumbel):
    n = z.shape[0]
    c_sqr = jnp.sum(codebook ** 2, axis=1)[None, :]
    z_sqr = jnp.sum(z ** 2, axis=1, keepdims=True)
    z2 = z + z
    grid = (n // BN, K_SLOTS // BK)
    zq, idx, kl, cl = pl.pallas_call(
        _vq_body,
        grid=grid,
        in_specs=[
            pl.BlockSpec((BN, D), lambda i, j: (i, 0)),
            pl.BlockSpec((BK, D), lambda i, j: (j, 0)),
            pl.BlockSpec((1, BK), lambda i, j: (0, j)),
            pl.BlockSpec((BN, 1), lambda i, j: (i, 0)),
            pl.BlockSpec((BN, BK), lambda i, j: (i, j)),
        ],
        out_specs=[
            pl.BlockSpec((BN, D), lambda i, j: (i, 0)),
            pl.BlockSpec((BN, 1), lambda i, j: (i, 0)),
            pl.BlockSpec((1, 1), lambda i, j: (0, 0)),
            pl.BlockSpec((1, 1), lambda i, j: (0, 0)),
        ],
        out_shape=[
            jax.ShapeDtypeStruct((n, D), jnp.float32),
            jax.ShapeDtypeStruct((n, 1), jnp.int32),
            jax.ShapeDtypeStruct((1, 1), jnp.float32),
            jax.ShapeDtypeStruct((1, 1), jnp.float32),
        ],
        scratch_shapes=[
            pltpu.VMEM((BN, 1), jnp.float32),
            pltpu.VMEM((BN, 1), jnp.float32),
            pltpu.VMEM((BN, 1), jnp.float32),
            pltpu.VMEM((BN, 1), jnp.float32),
            pltpu.VMEM((BN, D), jnp.float32),
            pltpu.VMEM((BN, 1), jnp.float32),
            pltpu.VMEM((BN, 1), jnp.int32),
        ],
    )(z2, codebook, c_sqr, z_sqr, gumbel)
    return zq, idx, kl, cl


def kernel(z_e, codebook, gumbel):
    bs, feat, w, h = z_e.shape
    z = jnp.transpose(z_e, (0, 2, 3, 1)).reshape(bs * w * h, feat)
    zq_flat, idx_flat, kl, cl = _vq_call(z, codebook, gumbel)
    z_q = zq_flat.reshape(bs, w, h, feat).transpose(0, 3, 1, 2)
    hard = idx_flat.reshape(bs, w, h)
    return z_q, hard, kl[0, 0], cl[0, 0]


# speedup vs baseline: 1.1254x; 1.0821x over previous
import functools
import math

import jax
import jax.numpy as jnp
from jax.experimental import pallas as pl
from jax.experimental.pallas import tpu as pltpu

K_SLOTS = 8192
D = 256
LOG_K = math.log(K_SLOTS)
NEG_BIG = -1e30

BN = 1152
BK = 1024


def _vq_body(z2_ref, cb_ref, csq_ref, zsq_ref, g_ref,
             zq_ref, idx_ref, kl_ref, cl_ref,
             s1, t1, m2, s2, acc, bv, bi):
    i = pl.program_id(0)
    j = pl.program_id(1)
    nk = pl.num_programs(1)

    @pl.when(j == 0)
    def _init():
        s1[...] = jnp.zeros_like(s1)
        t1[...] = jnp.zeros_like(t1)
        m2[...] = jnp.full_like(m2, NEG_BIG)
        s2[...] = jnp.zeros_like(s2)
        acc[...] = jnp.zeros_like(acc)
        bv[...] = jnp.full_like(bv, NEG_BIG)
        bi[...] = jnp.zeros_like(bi)

    z2 = z2_ref[...]
    cb = cb_ref[...]
    g = g_ref[...]

    z_sqr = zsq_ref[...]
    c_sqr = csq_ref[...]
    zc2 = jax.lax.dot_general(
        z2, cb, (((1,), (1,)), ((), ())),
        preferred_element_type=jnp.float32)

    l = zc2 - (z_sqr + c_sqr)
    L = l + g

    bm2 = jnp.max(L, axis=1, keepdims=True)
    m2o = m2[...]
    m2n = jnp.maximum(m2o, bm2)
    dm = m2o - m2n
    a1 = jnp.exp(dm)
    a2 = a1 * a1

    u1 = l - m2n
    e1 = jnp.exp(u1)
    s1o = s1[...]
    s1[...] = a1 * s1o + jnp.sum(e1, axis=1, keepdims=True)
    t1[...] = a1 * (t1[...] + s1o * dm) + jnp.sum(
        e1 * u1, axis=1, keepdims=True)

    e2 = jnp.exp((L - m2n) * 2.0)
    s2[...] = a2 * s2[...] + jnp.sum(e2, axis=1, keepdims=True)
    acc[...] = a2 * acc[...] + jax.lax.dot_general(
        e2, cb, (((1,), (0,)), ((), ())),
        preferred_element_type=jnp.float32)
    m2[...] = m2n

    col = jax.lax.broadcasted_iota(jnp.int32, (BN, BK), 1)
    idx_blk = jnp.min(jnp.where(L == bm2, col, K_SLOTS),
                      axis=1, keepdims=True) + j * BK
    bvo = bv[...]
    take = bm2 > bvo
    bi[...] = jnp.where(take, idx_blk, bi[...])
    bv[...] = jnp.maximum(bm2, bvo)

    @pl.when(j == nk - 1)
    def _fin():
        s1f = s1[...]
        r = t1[...] / s1f
        kl_rows = r - jnp.log(s1f) + LOG_K
        cl_rows = -(r + m2[...])
        zq_ref[...] = acc[...] / s2[...]
        idx_ref[...] = bi[...]

        @pl.when(i == 0)
        def _zero():
            kl_ref[...] = jnp.zeros_like(kl_ref)
            cl_ref[...] = jnp.zeros_like(cl_ref)

        kl_ref[...] += jnp.sum(kl_rows, keepdims=True) / 8.0
        cl_ref[...] += jnp.sum(cl_rows, keepdims=True) / 8.0


@functools.partial(jax.jit, static_argnames=())
def _vq_call(z, codebook, gumbel):
    n = z.shape[0]
    c_sqr = jnp.sum(codebook ** 2, axis=1)[None, :]
    z_sqr = jnp.sum(z ** 2, axis=1, keepdims=True)
    z2 = z + z
    grid = (n // BN, K_SLOTS // BK)
    zq, idx, kl, cl = pl.pallas_call(
        _vq_body,
        grid=grid,
        in_specs=[
            pl.BlockSpec((BN, D), lambda i, j: (i, 0)),
            pl.BlockSpec((BK, D), lambda i, j: (j, 0)),
            pl.BlockSpec((1, BK), lambda i, j: (0, j)),
            pl.BlockSpec((BN, 1), lambda i, j: (i, 0)),
            pl.BlockSpec((BN, BK), lambda i, j: (i, j)),
        ],
        out_specs=[
            pl.BlockSpec((BN, D), lambda i, j: (i, 0)),
            pl.BlockSpec((BN, 1), lambda i, j: (i, 0)),
            pl.BlockSpec((1, 1), lambda i, j: (0, 0)),
            pl.BlockSpec((1, 1), lambda i, j: (0, 0)),
        ],
        out_shape=[
            jax.ShapeDtypeStruct((n, D), jnp.float32),
            jax.ShapeDtypeStruct((n, 1), jnp.int32),
            jax.ShapeDtypeStruct((1, 1), jnp.float32),
            jax.ShapeDtypeStruct((1, 1), jnp.float32),
        ],
        scratch_shapes=[
            pltpu.VMEM((BN, 1), jnp.float32),
            pltpu.VMEM((BN, 1), jnp.float32),
            pltpu.VMEM((BN, 1), jnp.float32),
            pltpu.VMEM((BN, 1), jnp.float32),
            pltpu.VMEM((BN, D), jnp.float32),
            pltpu.VMEM((BN, 1), jnp.float32),
            pltpu.VMEM((BN, 1), jnp.int32),
        ],
    )(z2, codebook, c_sqr, z_sqr, gumbel)
    return zq, idx, kl, cl


def kernel(z_e, codebook, gumbel):
    bs, feat, w, h = z_e.shape
    z = jnp.transpose(z_e, (0, 2, 3, 1)).reshape(bs * w * h, feat)
    zq_flat, idx_flat, kl, cl = _vq_call(z, codebook, gumbel)
    z_q = zq_flat.reshape(bs, w, h, feat).transpose(0, 3, 1, 2)
    hard = idx_flat.reshape(bs, w, h)
    return z_q, hard, kl[0, 0], cl[0, 0]
